# baseline (speedup 1.0000x reference)
import jax
import jax.numpy as jnp
from jax.experimental import pallas as pl
from jax.experimental.pallas import tpu as pltpu


def _pool_half(x, w, bias):
    logits = jnp.sum(x * w, axis=3, keepdims=True) + bias
    logits = logits - jnp.max(logits, axis=2, keepdims=True)
    e = jnp.exp(logits)
    p = e / jnp.sum(e, axis=2, keepdims=True)
    return jnp.sum(p * x, axis=2, keepdims=True)


def _seqpool_body(x0_ref, x1_ref, w_ref, b_ref, o_ref):
    w = w_ref[...]
    bias = b_ref[0]
    o_ref[0:1] = _pool_half(x0_ref[...], w, bias).astype(o_ref.dtype)
    o_ref[1:2] = _pool_half(x1_ref[...], w, bias).astype(o_ref.dtype)


def _seqpool_body_single(x0_ref, w_ref, b_ref, o_ref):
    o_ref[...] = _pool_half(x0_ref[...], w_ref[...], b_ref[0]).astype(o_ref.dtype)


def _pick_batch_tile(H):
    for tb in range(min(256, H), 0, -8):
        if H % tb == 0:
            return tb
    return 1


def kernel(x, w, b):
    B, N, D = x.shape
    if B % 2 == 0:
        halves, H = 2, B // 2
    else:
        halves, H = 1, B
    xr = x.reshape(halves, H, N, D)
    tb = _pick_batch_tile(H)
    grid = (H // tb,)
    in_specs = [
        pl.BlockSpec((1, tb, N, D), lambda i, h=h: (h, i, 0, 0))
        for h in range(halves)
    ]
    in_specs += [
        pl.BlockSpec(memory_space=pltpu.MemorySpace.VMEM),
        pl.BlockSpec(memory_space=pltpu.MemorySpace.SMEM),
    ]
    body = _seqpool_body if halves == 2 else _seqpool_body_single
    out = pl.pallas_call(
        body,
        out_shape=jax.ShapeDtypeStruct((halves, H, 1, D), x.dtype),
        grid=grid,
        in_specs=in_specs,
        out_specs=pl.BlockSpec((halves, tb, 1, D), lambda i: (0, i, 0, 0)),
        compiler_params=pltpu.CompilerParams(
            dimension_semantics=("parallel",),
            vmem_limit_bytes=64 * 1024 * 1024,
        ),
    )(*([xr] * halves), w, b)
    return out.reshape(B, 1, D)


# speedup vs baseline: 1.0200x; 1.0200x over previous
import jax
import jax.numpy as jnp
from jax.experimental import pallas as pl
from jax.experimental.pallas import tpu as pltpu


def _seqpool_body(x_ref, w_ref, b_ref, o_ref):
    x = x_ref[...]
    logits = jnp.sum(x * w_ref[...], axis=2, keepdims=True) + b_ref[0]
    logits = logits - jnp.max(logits, axis=1, keepdims=True)
    e = jnp.exp(logits)
    p = e / jnp.sum(e, axis=1, keepdims=True)
    o_ref[...] = jnp.sum(p * x, axis=1, keepdims=True).astype(o_ref.dtype)


def _pick_batch_tile(B):
    for tb in range(min(512, B), 0, -8):
        if B % tb == 0:
            return tb
    return 1


def kernel(x, w, b):
    B, N, D = x.shape
    tb = _pick_batch_tile(B)
    grid = (B // tb,)
    out = pl.pallas_call(
        _seqpool_body,
        out_shape=jax.ShapeDtypeStruct((B, 1, D), x.dtype),
        grid=grid,
        in_specs=[
            pl.BlockSpec((tb, N, D), lambda i: (i, 0, 0)),
            pl.BlockSpec(memory_space=pltpu.MemorySpace.VMEM),
            pl.BlockSpec(memory_space=pltpu.MemorySpace.SMEM),
        ],
        out_specs=pl.BlockSpec((tb, 1, D), lambda i: (i, 0, 0)),
        compiler_params=pltpu.CompilerParams(
            dimension_semantics=("parallel",),
            vmem_limit_bytes=64 * 1024 * 1024,
        ),
    )(x, w, b)
    return out
